# Initial kernel scaffold; baseline (speedup 1.0000x reference)
import jax
import jax.numpy as jnp
from jax.experimental import pallas as pl


def kernel(X, weights):
    raise NotImplementedError("write your pallas kernel here")


# speedup vs baseline: 1.1099x; 1.1099x over previous
import functools

import jax
import jax.numpy as jnp
from jax import lax
from jax.experimental import pallas as pl
from jax.experimental.pallas import tpu as pltpu
from jax.experimental.pallas import tpu_sc as plsc

INPUT_SIZE = 1000000
OUTPUT_SIZE = 32
BATCH = 16384
HIST = 50

B = BATCH * HIST
NC = 2
NS = 16
NW = NC * NS
B_PER_W = B // NW
IDX_MINOR = 128
N_IDX_ROWS = B_PER_W // IDX_MINOR
K = 20
CHUNK = K * IDX_MINOR
N_CHUNKS = B_PER_W // CHUNK


def _gather_kernel(table_hbm, idx_hbm, out_hbm, idx_v, rows_v, sem):
    wid = lax.axis_index("s") * NC + lax.axis_index("c")
    base = wid * B_PER_W

    pltpu.sync_copy(idx_hbm.at[wid], idx_v)

    def body(g, carry):
        copies = []
        for j in range(K):
            c = pltpu.make_async_copy(
                table_hbm.at[idx_v.at[g * K + j]],
                rows_v.at[pl.ds(j * IDX_MINOR, IDX_MINOR)],
                sem,
            )
            c.start()
            copies.append(c)
        for c in copies:
            c.wait()
        pltpu.sync_copy(rows_v, out_hbm.at[pl.ds(base + g * CHUNK, CHUNK)])
        return carry

    lax.fori_loop(0, N_CHUNKS, body, 0, unroll=False)


@jax.jit
def _embedded(idx_grouped, weights):
    mesh = plsc.VectorSubcoreMesh(core_axis_name="c", subcore_axis_name="s")
    run = functools.partial(
        pl.kernel,
        mesh=mesh,
        out_type=jax.ShapeDtypeStruct((B, OUTPUT_SIZE), jnp.float32),
        scratch_types=[
            pltpu.VMEM((N_IDX_ROWS, IDX_MINOR), jnp.int32),
            pltpu.VMEM((CHUNK, OUTPUT_SIZE), jnp.float32),
            pltpu.SemaphoreType.DMA,
        ],
        compiler_params=pltpu.CompilerParams(use_tc_tiling_on_sc=False),
    )(_gather_kernel)
    return run(weights, idx_grouped)


def kernel(X, weights):
    idx_grouped = X.reshape(-1).astype(jnp.int32).reshape(NW, N_IDX_ROWS, IDX_MINOR)
    out = _embedded(idx_grouped, weights)
    return out.reshape(BATCH, HIST, OUTPUT_SIZE)
